# Initial kernel scaffold; baseline (speedup 1.0000x reference)
import functools

import jax
import jax.numpy as jnp
from jax import lax
from jax.experimental import pallas as pl
from jax.experimental.pallas import tpu as pltpu
from jax.experimental.pallas import tpu_sc as plsc

_NC = 2
_NS = 16
_ACC_ROWS = 51200


def _mm_body(x_ref, w_ref, o_ref):
    o_ref[...] = lax.dot_general(
        x_ref[...], w_ref[...], (((1,), (0,)), ((), ())),
        preferred_element_type=jnp.float32, precision=lax.Precision.HIGHEST)


def _tc_matmul(x, w):
    n, d = x.shape
    r = 2000
    return pl.pallas_call(
        _mm_body,
        grid=(n // r,),
        in_specs=[pl.BlockSpec((r, d), lambda i: (i, 0)),
                  pl.BlockSpec((d, d), lambda i: (0, 0))],
        out_specs=pl.BlockSpec((r, d), lambda i: (i, 0)),
        out_shape=jax.ShapeDtypeStruct((n, d), jnp.float32),
    )(x, w)


def _norm_body(p_ref, h_ref, dinv_ref, t_ref):
    p = p_ref[...]
    deg = p[0, :, 0:1] + p[1, :, 0:1] + 1.0
    dinv = lax.rsqrt(deg)
    dinv_ref[...] = dinv
    h = h_ref[...] * dinv
    t_ref[0, :, :] = h[:, :32]
    t_ref[1, :, :] = h[:, 32:]


def _tc_norm(pdeg, h1):
    n, d = h1.shape
    r = 2000
    return pl.pallas_call(
        _norm_body,
        grid=(n // r,),
        in_specs=[pl.BlockSpec((2, r, 16), lambda i: (0, i, 0)),
                  pl.BlockSpec((r, d), lambda i: (i, 0))],
        out_specs=(pl.BlockSpec((r, 1), lambda i: (i, 0)),
                   pl.BlockSpec((2, r, 32), lambda i: (0, i, 0))),
        out_shape=(jax.ShapeDtypeStruct((n, 1), jnp.float32),
                   jax.ShapeDtypeStruct((2, n, 32), jnp.float32)),
    )(pdeg, h1)


def _combine_body(y_ref, h_ref, dinv_ref, b_ref, w_ref, h2_ref, t_ref):
    d = dinv_ref[...]
    y = jnp.concatenate([y_ref[0], y_ref[1]], axis=1)
    z = jnp.maximum(y * d + h_ref[...] * (d * d) + b_ref[...], 0.0)
    h2 = lax.dot_general(
        z, w_ref[...], (((1,), (0,)), ((), ())),
        preferred_element_type=jnp.float32, precision=lax.Precision.HIGHEST)
    h2_ref[...] = h2
    ht = h2 * d
    t_ref[0, :, :] = ht[:, :32]
    t_ref[1, :, :] = ht[:, 32:]


def _tc_combine(y1, h1, dinv, b1, w2):
    n, d = h1.shape
    r = 2000
    return pl.pallas_call(
        _combine_body,
        grid=(n // r,),
        in_specs=[pl.BlockSpec((2, r, 32), lambda i: (0, i, 0)),
                  pl.BlockSpec((r, d), lambda i: (i, 0)),
                  pl.BlockSpec((r, 1), lambda i: (i, 0)),
                  pl.BlockSpec((1, d), lambda i: (0, 0)),
                  pl.BlockSpec((d, d), lambda i: (0, 0))],
        out_specs=(pl.BlockSpec((r, d), lambda i: (i, 0)),
                   pl.BlockSpec((2, r, 32), lambda i: (0, i, 0))),
        out_shape=(jax.ShapeDtypeStruct((n, d), jnp.float32),
                   jax.ShapeDtypeStruct((2, n, 32), jnp.float32)),
    )(y1, h1, dinv, b1, w2)


def _final_body(y_ref, h_ref, dinv_ref, b_ref, o_ref):
    d = dinv_ref[...]
    y = jnp.concatenate([y_ref[0], y_ref[1]], axis=1)
    o_ref[...] = jnp.maximum(y * d + h_ref[...] * (d * d) + b_ref[...], 0.0)


def _tc_final(y2, h2, dinv, b2):
    n, d = h2.shape
    r = 2000
    return pl.pallas_call(
        _final_body,
        grid=(n // r,),
        in_specs=[pl.BlockSpec((2, r, 32), lambda i: (0, i, 0)),
                  pl.BlockSpec((r, d), lambda i: (i, 0)),
                  pl.BlockSpec((r, 1), lambda i: (i, 0)),
                  pl.BlockSpec((1, d), lambda i: (0, 0))],
        out_specs=pl.BlockSpec((r, d), lambda i: (i, 0)),
        out_shape=jax.ShapeDtypeStruct((n, d), jnp.float32),
    )(y2, h2, dinv, b2)


def _sc_degree(dst3, n_nodes):
    k_ch = dst3.shape[1]
    stripe = _ACC_ROWS // _NS
    rows = n_nodes // _NS
    mesh = plsc.VectorSubcoreMesh(core_axis_name="c", subcore_axis_name="s")

    @functools.partial(
        pl.kernel, mesh=mesh,
        out_type=jax.ShapeDtypeStruct((2 * n_nodes, 16), jnp.float32),
        scratch_types=[
            pltpu.VMEM((k_ch, 128), jnp.int32),
            pltpu.VMEM((128, 16), jnp.float32),
            pltpu.VMEM((128, 16), jnp.float32),
            pltpu.VMEM_SHARED((_ACC_ROWS, 16), jnp.float32),
            pltpu.SemaphoreType.DMA,
        ])
    def deg_kernel(dst_hbm, out_hbm, dst_v, ones_v, zero_v, acc, sem):
        c = lax.axis_index("c")
        s = lax.axis_index("s")
        w = c * _NS + s
        cp = pltpu.async_copy(dst_hbm.at[w], dst_v, sem)

        @pl.loop(0, 128)
        def _(i):
            ones_v[i, :] = jnp.ones((16,), jnp.float32)
            zero_v[i, :] = jnp.zeros((16,), jnp.float32)

        @pl.loop(0, stripe // 128)
        def _(j):
            pltpu.sync_copy(zero_v, acc.at[pl.ds(s * stripe + j * 128, 128)])

        cp.wait()
        plsc.subcore_barrier()

        @pl.loop(0, k_ch)
        def _(j):
            pltpu.sync_copy(ones_v, acc.at[dst_v.at[j]], add=True)

        plsc.subcore_barrier()
        pltpu.sync_copy(acc.at[pl.ds(s * rows, rows)],
                        out_hbm.at[pl.ds(c * n_nodes + s * rows, rows)])

    return deg_kernel(dst3)


def _sc_gather_scatter(table, src4, dst3, n_nodes):
    k_ch = dst3.shape[1]
    stripe = _ACC_ROWS // _NS
    rows = n_nodes // _NS
    mesh = plsc.VectorSubcoreMesh(core_axis_name="c", subcore_axis_name="s")

    @functools.partial(
        pl.kernel, mesh=mesh,
        out_type=jax.ShapeDtypeStruct((2 * n_nodes, 32), jnp.float32),
        scratch_types=[
            pltpu.VMEM((k_ch, 128), jnp.int32),
            pltpu.VMEM((k_ch, 128), jnp.int32),
            pltpu.VMEM((2, 128, 32), jnp.float32),
            pltpu.VMEM_SHARED((_ACC_ROWS, 32), jnp.float32),
            pltpu.SemaphoreType.DMA,
            pltpu.SemaphoreType.DMA,
        ])
    def msg_kernel(tab_hbm, src_hbm, dst_hbm, out_hbm,
                   src_v, dst_v, gbuf, acc, sem0, sem1):
        c = lax.axis_index("c")
        s = lax.axis_index("s")
        cp0 = pltpu.async_copy(src_hbm.at[c].at[s], src_v, sem0)
        cp1 = pltpu.async_copy(dst_hbm.at[s], dst_v, sem1)

        @pl.loop(0, 128)
        def _(i):
            gbuf[0, i, pl.ds(0, 16)] = jnp.zeros((16,), jnp.float32)
            gbuf[0, i, pl.ds(16, 16)] = jnp.zeros((16,), jnp.float32)

        @pl.loop(0, stripe // 128)
        def _(j):
            pltpu.sync_copy(gbuf.at[0], acc.at[pl.ds(s * stripe + j * 128, 128)])

        cp0.wait()
        cp1.wait()
        plsc.subcore_barrier()

        @pl.loop(0, k_ch, step=2)
        def _(j):
            ga = pltpu.async_copy(tab_hbm.at[src_v.at[j]], gbuf.at[0], sem0)
            gb = pltpu.async_copy(tab_hbm.at[src_v.at[j + 1]], gbuf.at[1], sem1)
            ga.wait()
            pltpu.sync_copy(gbuf.at[0], acc.at[dst_v.at[j]], add=True)
            gb.wait()
            pltpu.sync_copy(gbuf.at[1], acc.at[dst_v.at[j + 1]], add=True)

        plsc.subcore_barrier()
        pltpu.sync_copy(acc.at[pl.ds(s * rows, rows)],
                        out_hbm.at[pl.ds(c * n_nodes + s * rows, rows)])

    return msg_kernel(table, src4, dst3)


def kernel(x, edge_index, W1, b1, W2, b2):
    n, d = x.shape
    e = edge_index.shape[1]
    src = edge_index[0]
    dst = edge_index[1]

    ep = -(-e // 8192) * 8192
    pad = ep - e
    srcp = jnp.concatenate([src, jnp.zeros((pad,), jnp.int32)])
    dstp = jnp.concatenate([dst, jnp.full((pad,), n, jnp.int32)])
    dst_deg = dstp.reshape(2 * _NS, ep // 4096, 128)
    dst_m = dstp.reshape(_NS, ep // 2048, 128)
    src_m = srcp.reshape(_NS, ep // 2048, 128)
    src4 = jnp.stack([src_m, src_m + n])

    h1 = _tc_matmul(x, W1)
    pdeg = _sc_degree(dst_deg, n)
    dinv, t1 = _tc_norm(pdeg.reshape(2, n, 16), h1)
    y1 = _sc_gather_scatter(t1.reshape(2 * n, 32), src4, dst_m, n)
    h2, t2 = _tc_combine(y1.reshape(2, n, 32), h1, dinv,
                         b1.reshape(1, d), W2)
    y2 = _sc_gather_scatter(t2.reshape(2 * n, 32), src4, dst_m, n)
    return _tc_final(y2.reshape(2, n, 32), h2, dinv, b2.reshape(1, d))


# speedup vs baseline: 5.0691x; 5.0691x over previous
import functools

import jax
import jax.numpy as jnp
from jax import lax
from jax.experimental import pallas as pl
from jax.experimental.pallas import tpu as pltpu
from jax.experimental.pallas import tpu_sc as plsc

_NC = 2
_NS = 16
_AR2 = 26624
_WR2 = 1568
_NR2 = _NS * _WR2


def _mm_body(x_ref, w_ref, o_ref):
    o_ref[...] = lax.dot_general(
        x_ref[...], w_ref[...], (((1,), (0,)), ((), ())),
        preferred_element_type=jnp.float32, precision=lax.Precision.HIGHEST)


def _tc_matmul(x, w):
    n, d = x.shape
    r = 2000
    return pl.pallas_call(
        _mm_body,
        grid=(n // r,),
        in_specs=[pl.BlockSpec((r, d), lambda i: (i, 0)),
                  pl.BlockSpec((d, d), lambda i: (0, 0))],
        out_specs=pl.BlockSpec((r, d), lambda i: (i, 0)),
        out_shape=jax.ShapeDtypeStruct((n, d), jnp.float32),
    )(x, w)


def _norm_body(p_ref, h_ref, dinv_ref, t_ref):
    deg = p_ref[0, :, 0:1] + 1.0
    dinv = lax.rsqrt(deg)
    dinv_ref[...] = dinv
    h = h_ref[...] * dinv
    for q in range(4):
        t_ref[q, :, :] = h[:, 16 * q:16 * (q + 1)]


def _tc_norm(pdeg, h1):
    n, d = h1.shape
    r = 1000
    hb = (n // 2) // r
    return pl.pallas_call(
        _norm_body,
        grid=(n // r,),
        in_specs=[pl.BlockSpec((1, r, 16), lambda i: (i // hb, i % hb, 0)),
                  pl.BlockSpec((r, d), lambda i: (i, 0))],
        out_specs=(pl.BlockSpec((r, 1), lambda i: (i, 0)),
                   pl.BlockSpec((4, r, 16), lambda i: (0, i, 0))),
        out_shape=(jax.ShapeDtypeStruct((n, 1), jnp.float32),
                   jax.ShapeDtypeStruct((4, n, 16), jnp.float32)),
    )(pdeg, h1)


def _step_body(y_ref, h_ref, dinv_ref, b_ref, w_ref, z_ref, h2_ref, t_ref):
    d = dinv_ref[...]
    y = jnp.concatenate([y_ref[0], y_ref[1], y_ref[2], y_ref[3]], axis=1)
    z = jnp.maximum(y * d + h_ref[...] * (d * d) + b_ref[...], 0.0)
    z_ref[...] = z
    h2 = lax.dot_general(
        z, w_ref[...], (((1,), (0,)), ((), ())),
        preferred_element_type=jnp.float32, precision=lax.Precision.HIGHEST)
    h2_ref[...] = h2
    ht = h2 * d
    for q in range(4):
        t_ref[q, :, :] = ht[:, 16 * q:16 * (q + 1)]


def _tc_step(y, h, dinv, b, w):
    n, d = h.shape
    r = 1000
    hb = (n // 2) // r
    return pl.pallas_call(
        _step_body,
        grid=(n // r,),
        in_specs=[pl.BlockSpec((4, r, 16), lambda i: (i // hb, i % hb, 0)),
                  pl.BlockSpec((r, d), lambda i: (i, 0)),
                  pl.BlockSpec((r, 1), lambda i: (i, 0)),
                  pl.BlockSpec((1, d), lambda i: (0, 0)),
                  pl.BlockSpec((d, d), lambda i: (0, 0))],
        out_specs=(pl.BlockSpec((r, d), lambda i: (i, 0)),
                   pl.BlockSpec((r, d), lambda i: (i, 0)),
                   pl.BlockSpec((4, r, 16), lambda i: (0, i, 0))),
        out_shape=(jax.ShapeDtypeStruct((n, d), jnp.float32),
                   jax.ShapeDtypeStruct((n, d), jnp.float32),
                   jax.ShapeDtypeStruct((4, n, 16), jnp.float32)),
    )(y, h, dinv, b, w)


def _sc_degree(dst4, n_nodes):
    k_ch = dst4.shape[2]
    stripe = _AR2 // _NS
    mesh = plsc.VectorSubcoreMesh(core_axis_name="c", subcore_axis_name="s")

    @functools.partial(
        pl.kernel, mesh=mesh,
        compiler_params=pltpu.CompilerParams(use_tc_tiling_on_sc=False),
        out_type=jax.ShapeDtypeStruct((2 * _NR2, 16), jnp.float32),
        scratch_types=[
            pltpu.VMEM((k_ch, 128), jnp.int32),
            pltpu.VMEM((128, 16), jnp.float32),
            pltpu.VMEM((128, 16), jnp.float32),
            pltpu.VMEM_SHARED((_AR2, 16), jnp.float32),
            pltpu.SemaphoreType.DMA,
        ])
    def deg_kernel(dst_hbm, out_hbm, dst_v, ones_v, zero_v, acc, sem):
        c = lax.axis_index("c")
        s = lax.axis_index("s")
        cp = pltpu.async_copy(dst_hbm.at[c].at[s], dst_v, sem)

        @pl.loop(0, 128)
        def _(i):
            ones_v[i, :] = jnp.ones((16,), jnp.float32)
            zero_v[i, :] = jnp.zeros((16,), jnp.float32)

        @pl.loop(0, stripe // 128)
        def _(j):
            pltpu.sync_copy(zero_v, acc.at[pl.ds(s * stripe + j * 128, 128)])

        cp.wait()
        plsc.subcore_barrier()

        @pl.loop(0, k_ch)
        def _(j):
            pltpu.sync_copy(ones_v, acc.at[dst_v.at[j]], add=True)

        plsc.subcore_barrier()
        pltpu.sync_copy(acc.at[pl.ds(s * _WR2, _WR2)],
                        out_hbm.at[pl.ds(c * _NR2 + s * _WR2, _WR2)])

    return deg_kernel(dst4)


def _sc_gather_scatter(table, src3, dst4, n_nodes):
    k_ch = src3.shape[1]
    stripe = _AR2 // _NS
    mesh = plsc.VectorSubcoreMesh(core_axis_name="c", subcore_axis_name="s")

    @functools.partial(
        pl.kernel, mesh=mesh,
        compiler_params=pltpu.CompilerParams(use_tc_tiling_on_sc=False),
        out_type=jax.ShapeDtypeStruct((8 * _NR2, 16), jnp.float32),
        scratch_types=[
            pltpu.VMEM((k_ch, 128), jnp.int32),
            pltpu.VMEM((k_ch, 128), jnp.int32),
            pltpu.VMEM((2, 128, 16), jnp.float32),
            pltpu.VMEM_SHARED((_AR2, 16), jnp.float32),
            pltpu.SemaphoreType.DMA,
            pltpu.SemaphoreType.DMA,
        ])
    def msg_kernel(tab_hbm, src_hbm, dst_hbm, out_hbm,
                   src_v, dst_v, gbuf, acc, sem0, sem1):
        c = lax.axis_index("c")
        s = lax.axis_index("s")
        cp0 = pltpu.async_copy(src_hbm.at[s], src_v, sem0)
        cp1 = pltpu.async_copy(dst_hbm.at[c].at[s], dst_v, sem1)
        cp0.wait()
        cp1.wait()

        for q in range(4):
            if q:
                plsc.subcore_barrier()

                @pl.loop(0, k_ch)
                def _(j):
                    for i in range(8):
                        sl = pl.ds(16 * i, 16)
                        src_v[j, sl] = src_v[j, sl] + n_nodes

            @pl.loop(0, 128)
            def _(i):
                gbuf[0, i, :] = jnp.zeros((16,), jnp.float32)

            @pl.loop(0, stripe // 128)
            def _(j):
                pltpu.sync_copy(gbuf.at[0],
                                acc.at[pl.ds(s * stripe + j * 128, 128)])

            plsc.subcore_barrier()

            @pl.loop(0, k_ch, step=2)
            def _(j):
                ga = pltpu.async_copy(tab_hbm.at[src_v.at[j]], gbuf.at[0], sem0)
                gb = pltpu.async_copy(tab_hbm.at[src_v.at[j + 1]], gbuf.at[1],
                                      sem1)
                ga.wait()
                pltpu.sync_copy(gbuf.at[0], acc.at[dst_v.at[j]], add=True)
                gb.wait()
                pltpu.sync_copy(gbuf.at[1], acc.at[dst_v.at[j + 1]], add=True)

            plsc.subcore_barrier()
            pltpu.sync_copy(
                acc.at[pl.ds(s * _WR2, _WR2)],
                out_hbm.at[pl.ds((c * 4 + q) * _NR2 + s * _WR2, _WR2)])

    return msg_kernel(table, src3, dst4)


def kernel(x, edge_index, W1, b1, W2, b2):
    n, d = x.shape
    e = edge_index.shape[1]
    src = edge_index[0]
    dst = edge_index[1]
    nh = n // 2

    ep = -(-e // 8192) * 8192
    pad = ep - e
    srcp = jnp.concatenate([src, jnp.zeros((pad,), jnp.int32)])
    dstp = jnp.concatenate([dst, jnp.full((pad,), n, jnp.int32)])
    src_m = srcp.reshape(_NS, ep // 2048, 128)
    dl0 = jnp.where(dstp < nh, dstp, _NR2)
    dl1 = jnp.where((dstp >= nh) & (dstp < n), dstp - nh, _NR2)
    dst4 = jnp.stack([dl0.reshape(_NS, ep // 2048, 128),
                      dl1.reshape(_NS, ep // 2048, 128)])

    h1 = _tc_matmul(x, W1)
    pdeg = _sc_degree(dst4, n)
    dinv, t1 = _tc_norm(pdeg.reshape(2, _NR2, 16), h1)

    def body(carry, bw):
        table, h = carry
        b_i, w_i = bw
        y = _sc_gather_scatter(table, src_m, dst4, n)
        z, h2, t2 = _tc_step(y.reshape(8, _NR2, 16), h, dinv, b_i, w_i)
        return (t2.reshape(4 * n, 16), h2), z

    bs = jnp.stack([b1.reshape(1, d), b2.reshape(1, d)])
    ws = jnp.stack([W2, W2])
    _, zs = lax.scan(body, (t1.reshape(4 * n, 16), h1), (bs, ws))
    return zs[1]


# speedup vs baseline: 5.1480x; 1.0156x over previous
import functools

import jax
import jax.numpy as jnp
from jax import lax
from jax.experimental import pallas as pl
from jax.experimental.pallas import tpu as pltpu
from jax.experimental.pallas import tpu_sc as plsc

_NC = 2
_NS = 16
_AR2 = 26624
_WR2 = 1568
_NR2 = _NS * _WR2


def _mm_body(x_ref, w_ref, o_ref):
    o_ref[...] = lax.dot_general(
        x_ref[...], w_ref[...], (((1,), (0,)), ((), ())),
        preferred_element_type=jnp.float32, precision=lax.Precision.HIGHEST)


def _tc_matmul(x, w):
    n, d = x.shape
    r = 2000
    return pl.pallas_call(
        _mm_body,
        grid=(n // r,),
        in_specs=[pl.BlockSpec((r, d), lambda i: (i, 0)),
                  pl.BlockSpec((d, d), lambda i: (0, 0))],
        out_specs=pl.BlockSpec((r, d), lambda i: (i, 0)),
        out_shape=jax.ShapeDtypeStruct((n, d), jnp.float32),
    )(x, w)


def _norm_body(p_ref, h_ref, dinv_ref, t_ref):
    deg = p_ref[0, :, 0:1] + 1.0
    dinv = lax.rsqrt(deg)
    dinv_ref[...] = dinv
    h = h_ref[...] * dinv
    for q in range(4):
        t_ref[q, :, :] = h[:, 16 * q:16 * (q + 1)]


def _tc_norm(pdeg, h1):
    n, d = h1.shape
    r = 1000
    hb = (n // 2) // r
    return pl.pallas_call(
        _norm_body,
        grid=(n // r,),
        in_specs=[pl.BlockSpec((1, r, 16), lambda i: (i // hb, i % hb, 0)),
                  pl.BlockSpec((r, d), lambda i: (i, 0))],
        out_specs=(pl.BlockSpec((r, 1), lambda i: (i, 0)),
                   pl.BlockSpec((4, r, 16), lambda i: (0, i, 0))),
        out_shape=(jax.ShapeDtypeStruct((n, 1), jnp.float32),
                   jax.ShapeDtypeStruct((4, n, 16), jnp.float32)),
    )(pdeg, h1)


def _step_body(y_ref, h_ref, dinv_ref, b_ref, w_ref, z_ref, h2_ref, t_ref):
    d = dinv_ref[...]
    y = jnp.concatenate([y_ref[0], y_ref[1], y_ref[2], y_ref[3]], axis=1)
    z = jnp.maximum(y * d + h_ref[...] * (d * d) + b_ref[...], 0.0)
    z_ref[...] = z
    h2 = lax.dot_general(
        z, w_ref[...], (((1,), (0,)), ((), ())),
        preferred_element_type=jnp.float32, precision=lax.Precision.HIGHEST)
    h2_ref[...] = h2
    ht = h2 * d
    for q in range(4):
        t_ref[q, :, :] = ht[:, 16 * q:16 * (q + 1)]


def _tc_step(y, h, dinv, b, w):
    n, d = h.shape
    r = 1000
    hb = (n // 2) // r
    return pl.pallas_call(
        _step_body,
        grid=(n // r,),
        in_specs=[pl.BlockSpec((4, r, 16), lambda i: (i // hb, i % hb, 0)),
                  pl.BlockSpec((r, d), lambda i: (i, 0)),
                  pl.BlockSpec((r, 1), lambda i: (i, 0)),
                  pl.BlockSpec((1, d), lambda i: (0, 0)),
                  pl.BlockSpec((d, d), lambda i: (0, 0))],
        out_specs=(pl.BlockSpec((r, d), lambda i: (i, 0)),
                   pl.BlockSpec((r, d), lambda i: (i, 0)),
                   pl.BlockSpec((4, r, 16), lambda i: (0, i, 0))),
        out_shape=(jax.ShapeDtypeStruct((n, d), jnp.float32),
                   jax.ShapeDtypeStruct((n, d), jnp.float32),
                   jax.ShapeDtypeStruct((4, n, 16), jnp.float32)),
    )(y, h, dinv, b, w)


def _sc_degree(dst4, n_nodes):
    k_ch = dst4.shape[2]
    stripe = _AR2 // _NS
    mesh = plsc.VectorSubcoreMesh(core_axis_name="c", subcore_axis_name="s")

    @functools.partial(
        pl.kernel, mesh=mesh,
        compiler_params=pltpu.CompilerParams(use_tc_tiling_on_sc=False),
        out_type=jax.ShapeDtypeStruct((2 * _NR2, 16), jnp.float32),
        scratch_types=[
            pltpu.VMEM((k_ch, 128), jnp.int32),
            pltpu.VMEM((128, 16), jnp.float32),
            pltpu.VMEM((128, 16), jnp.float32),
            pltpu.VMEM_SHARED((_AR2, 16), jnp.float32),
            pltpu.SemaphoreType.DMA,
        ])
    def deg_kernel(dst_hbm, out_hbm, dst_v, ones_v, zero_v, acc, sem):
        c = lax.axis_index("c")
        s = lax.axis_index("s")
        cp = pltpu.async_copy(dst_hbm.at[c].at[s], dst_v, sem)

        @pl.loop(0, 128)
        def _(i):
            ones_v[i, :] = jnp.ones((16,), jnp.float32)
            zero_v[i, :] = jnp.zeros((16,), jnp.float32)

        @pl.loop(0, stripe // 128)
        def _(j):
            pltpu.sync_copy(zero_v, acc.at[pl.ds(s * stripe + j * 128, 128)])

        cp.wait()
        plsc.subcore_barrier()

        @pl.loop(0, k_ch)
        def _(j):
            pltpu.sync_copy(ones_v, acc.at[dst_v.at[j]], add=True)

        plsc.subcore_barrier()
        pltpu.sync_copy(acc.at[pl.ds(s * _WR2, _WR2)],
                        out_hbm.at[pl.ds(c * _NR2 + s * _WR2, _WR2)])

    return deg_kernel(dst4)


def _sc_gather_scatter(table, src3, dst4, n_nodes):
    k_ch = src3.shape[1]
    nb = 8
    bch = k_ch // nb
    stripe = _AR2 // _NS
    mesh = plsc.VectorSubcoreMesh(core_axis_name="c", subcore_axis_name="s")

    @functools.partial(
        pl.kernel, mesh=mesh,
        compiler_params=pltpu.CompilerParams(use_tc_tiling_on_sc=False),
        out_type=jax.ShapeDtypeStruct((8 * _NR2, 16), jnp.float32),
        scratch_types=[
            pltpu.VMEM((2, bch, 128), jnp.int32),
            pltpu.VMEM((2, bch, 128), jnp.int32),
            pltpu.VMEM((7, 128, 16), jnp.float32),
            pltpu.VMEM_SHARED((_AR2, 16), jnp.float32),
            pltpu.SemaphoreType.DMA,
            pltpu.SemaphoreType.DMA,
            pltpu.SemaphoreType.DMA,
        ])
    def msg_kernel(tab_hbm, src_hbm, dst_hbm, out_hbm,
                   src_i, dst_i, gbuf, acc, semi_s, semi_d, semg):
        c = lax.axis_index("c")
        s = lax.axis_index("s")

        for q in range(4):
            if q:
                plsc.subcore_barrier()
            tab_q = tab_hbm.at[pl.ds(q * n_nodes, n_nodes)]

            pltpu.async_copy(src_hbm.at[s].at[pl.ds(0, bch)],
                             src_i.at[0], semi_s)
            pltpu.async_copy(dst_hbm.at[c].at[s].at[pl.ds(0, bch)],
                             dst_i.at[0], semi_d)

            @pl.loop(0, 128)
            def _(i):
                gbuf[0, i, :] = jnp.zeros((16,), jnp.float32)

            @pl.loop(0, stripe // 128)
            def _(j):
                pltpu.sync_copy(gbuf.at[0],
                                acc.at[pl.ds(s * stripe + j * 128, 128)])

            plsc.subcore_barrier()

            @pl.loop(0, nb)
            def _(blk):
                par = lax.rem(blk, 2)
                pltpu.make_async_copy(
                    src_hbm.at[s].at[pl.ds(blk * bch, bch)],
                    src_i.at[par], semi_s).wait()
                pltpu.make_async_copy(
                    dst_hbm.at[c].at[s].at[pl.ds(blk * bch, bch)],
                    dst_i.at[par], semi_d).wait()

                @pl.when(blk < nb - 1)
                def _():
                    pltpu.async_copy(
                        src_hbm.at[s].at[pl.ds((blk + 1) * bch, bch)],
                        src_i.at[1 - par], semi_s)
                    pltpu.async_copy(
                        dst_hbm.at[c].at[s].at[pl.ds((blk + 1) * bch, bch)],
                        dst_i.at[1 - par], semi_d)

                @pl.loop(0, bch, step=7)
                def _(jj):
                    gs = [pltpu.async_copy(
                              tab_q.at[src_i.at[par].at[jj + b]],
                              gbuf.at[b], semg)
                          for b in range(7)]
                    for b in range(7):
                        gs[b].wait()
                        pltpu.sync_copy(gbuf.at[b],
                                        acc.at[dst_i.at[par].at[jj + b]],
                                        add=True)

            plsc.subcore_barrier()
            pltpu.sync_copy(
                acc.at[pl.ds(s * _WR2, _WR2)],
                out_hbm.at[pl.ds((c * 4 + q) * _NR2 + s * _WR2, _WR2)])

    return msg_kernel(table, src3, dst4)


def kernel(x, edge_index, W1, b1, W2, b2):
    n, d = x.shape
    e = edge_index.shape[1]
    src = edge_index[0]
    dst = edge_index[1]
    nh = n // 2

    ep = -(-e // 8192) * 8192
    pad = ep - e
    srcp = jnp.concatenate([src, jnp.zeros((pad,), jnp.int32)])
    dstp = jnp.concatenate([dst, jnp.full((pad,), n, jnp.int32)])
    src_m = srcp.reshape(_NS, ep // 2048, 128)
    dl0 = jnp.where(dstp < nh, dstp, _NR2)
    dl1 = jnp.where((dstp >= nh) & (dstp < n), dstp - nh, _NR2)
    dst4 = jnp.stack([dl0.reshape(_NS, ep // 2048, 128),
                      dl1.reshape(_NS, ep // 2048, 128)])

    h1 = _tc_matmul(x, W1)
    pdeg = _sc_degree(dst4, n)
    dinv, t1 = _tc_norm(pdeg.reshape(2, _NR2, 16), h1)

    def body(carry, bw):
        table, h = carry
        b_i, w_i = bw
        y = _sc_gather_scatter(table, src_m, dst4, n)
        z, h2, t2 = _tc_step(y.reshape(8, _NR2, 16), h, dinv, b_i, w_i)
        return (t2.reshape(4 * n, 16), h2), z

    bs = jnp.stack([b1.reshape(1, d), b2.reshape(1, d)])
    ws = jnp.stack([W2, W2])
    _, zs = lax.scan(body, (t1.reshape(4 * n, 16), h1), (bs, ws))
    return zs[1]


# speedup vs baseline: 8.4405x; 1.6396x over previous
import functools

import jax
import jax.numpy as jnp
from jax import lax
from jax.experimental import pallas as pl
from jax.experimental.pallas import tpu as pltpu
from jax.experimental.pallas import tpu_sc as plsc

_NC = 2
_NS = 16
_AR2 = 26624
_WR2 = 1568
_NR2 = _NS * _WR2


def _mm_body(x_ref, w_ref, o_ref):
    o_ref[...] = lax.dot_general(
        x_ref[...], w_ref[...], (((1,), (0,)), ((), ())),
        preferred_element_type=jnp.float32, precision=lax.Precision.HIGHEST)


def _tc_matmul(x, w):
    n, d = x.shape
    r = 2000
    return pl.pallas_call(
        _mm_body,
        grid=(n // r,),
        in_specs=[pl.BlockSpec((r, d), lambda i: (i, 0)),
                  pl.BlockSpec((d, d), lambda i: (0, 0))],
        out_specs=pl.BlockSpec((r, d), lambda i: (i, 0)),
        out_shape=jax.ShapeDtypeStruct((n, d), jnp.float32),
    )(x, w)


def _norm_body(p_ref, h_ref, dinv_ref, t_ref):
    deg = p_ref[0, :, 0:1] + 1.0
    dinv = lax.rsqrt(deg)
    dinv_ref[...] = dinv
    h = h_ref[...] * dinv
    t_ref[0, :, :] = h[:, :32]
    t_ref[1, :, :] = h[:, 32:]


def _tc_norm(pdeg, h1):
    n, d = h1.shape
    r = 1000
    hb = (n // 2) // r
    return pl.pallas_call(
        _norm_body,
        grid=(n // r,),
        in_specs=[pl.BlockSpec((1, r, 16), lambda i: (i // hb, i % hb, 0)),
                  pl.BlockSpec((r, d), lambda i: (i, 0))],
        out_specs=(pl.BlockSpec((r, 1), lambda i: (i, 0)),
                   pl.BlockSpec((2, r, 32), lambda i: (0, i, 0))),
        out_shape=(jax.ShapeDtypeStruct((n, 1), jnp.float32),
                   jax.ShapeDtypeStruct((2, n, 32), jnp.float32)),
    )(pdeg, h1)


def _step_body(y_ref, h_ref, dinv_ref, b_ref, w_ref, z_ref, h2_ref, t_ref):
    d = dinv_ref[...]
    y = jnp.concatenate([y_ref[0], y_ref[1]], axis=1)
    z = jnp.maximum(y * d + h_ref[...] * (d * d) + b_ref[...], 0.0)
    z_ref[...] = z
    h2 = lax.dot_general(
        z, w_ref[...], (((1,), (0,)), ((), ())),
        preferred_element_type=jnp.float32, precision=lax.Precision.HIGHEST)
    h2_ref[...] = h2
    ht = h2 * d
    t_ref[0, :, :] = ht[:, :32]
    t_ref[1, :, :] = ht[:, 32:]


def _tc_step(y, h, dinv, b, w):
    n, d = h.shape
    r = 1000
    hb = (n // 2) // r
    return pl.pallas_call(
        _step_body,
        grid=(n // r,),
        in_specs=[pl.BlockSpec((2, r, 32), lambda i: (i // hb, i % hb, 0)),
                  pl.BlockSpec((r, d), lambda i: (i, 0)),
                  pl.BlockSpec((r, 1), lambda i: (i, 0)),
                  pl.BlockSpec((1, d), lambda i: (0, 0)),
                  pl.BlockSpec((d, d), lambda i: (0, 0))],
        out_specs=(pl.BlockSpec((r, d), lambda i: (i, 0)),
                   pl.BlockSpec((r, d), lambda i: (i, 0)),
                   pl.BlockSpec((2, r, 32), lambda i: (0, i, 0))),
        out_shape=(jax.ShapeDtypeStruct((n, d), jnp.float32),
                   jax.ShapeDtypeStruct((n, d), jnp.float32),
                   jax.ShapeDtypeStruct((2, n, 32), jnp.float32)),
    )(y, h, dinv, b, w)


def _sc_degree(dst4, n_nodes):
    k_ch = dst4.shape[2]
    stripe = _AR2 // _NS
    mesh = plsc.VectorSubcoreMesh(core_axis_name="c", subcore_axis_name="s")

    @functools.partial(
        pl.kernel, mesh=mesh,
        compiler_params=pltpu.CompilerParams(use_tc_tiling_on_sc=False),
        out_type=jax.ShapeDtypeStruct((2 * _NR2, 16), jnp.float32),
        scratch_types=[
            pltpu.VMEM((k_ch, 128), jnp.int32),
            pltpu.VMEM((128, 16), jnp.float32),
            pltpu.VMEM((128, 16), jnp.float32),
            pltpu.VMEM_SHARED((_AR2, 16), jnp.float32),
            pltpu.SemaphoreType.DMA,
        ])
    def deg_kernel(dst_hbm, out_hbm, dst_v, ones_v, zero_v, acc, sem):
        c = lax.axis_index("c")
        s = lax.axis_index("s")
        cp = pltpu.async_copy(dst_hbm.at[c].at[s], dst_v, sem)

        @pl.loop(0, 128)
        def _(i):
            ones_v[i, :] = jnp.ones((16,), jnp.float32)
            zero_v[i, :] = jnp.zeros((16,), jnp.float32)

        @pl.loop(0, stripe // 128)
        def _(j):
            pltpu.sync_copy(zero_v, acc.at[pl.ds(s * stripe + j * 128, 128)])

        cp.wait()
        plsc.subcore_barrier()

        @pl.loop(0, k_ch)
        def _(j):
            pltpu.sync_copy(ones_v, acc.at[dst_v.at[j]], add=True)

        plsc.subcore_barrier()
        pltpu.sync_copy(acc.at[pl.ds(s * _WR2, _WR2)],
                        out_hbm.at[pl.ds(c * _NR2 + s * _WR2, _WR2)])

    return deg_kernel(dst4)


def _sc_gather_scatter(table, src3, dst4, n_nodes):
    k_ch = src3.shape[1]
    nb = 8
    bch = k_ch // nb
    stripe = _AR2 // _NS
    mesh = plsc.VectorSubcoreMesh(core_axis_name="c", subcore_axis_name="s")

    @functools.partial(
        pl.kernel, mesh=mesh,
        compiler_params=pltpu.CompilerParams(use_tc_tiling_on_sc=False),
        out_type=jax.ShapeDtypeStruct((4 * _NR2, 32), jnp.float32),
        scratch_types=[
            pltpu.VMEM((2, bch, 128), jnp.int32),
            pltpu.VMEM((2, bch, 128), jnp.int32),
            pltpu.VMEM((7, 128, 32), jnp.float32),
            pltpu.VMEM_SHARED((_AR2, 32), jnp.float32),
            pltpu.SemaphoreType.DMA,
            pltpu.SemaphoreType.DMA,
            pltpu.SemaphoreType.DMA,
        ])
    def msg_kernel(tab_hbm, src_hbm, dst_hbm, out_hbm,
                   src_i, dst_i, gbuf, acc, semi_s, semi_d, semg):
        c = lax.axis_index("c")
        s = lax.axis_index("s")

        for q in range(2):
            if q:
                plsc.subcore_barrier()
            tab_q = tab_hbm.at[pl.ds(q * n_nodes, n_nodes)]

            pltpu.async_copy(src_hbm.at[s].at[pl.ds(0, bch)],
                             src_i.at[0], semi_s)
            pltpu.async_copy(dst_hbm.at[c].at[s].at[pl.ds(0, bch)],
                             dst_i.at[0], semi_d)

            @pl.loop(0, 128)
            def _(i):
                gbuf[0, i, pl.ds(0, 16)] = jnp.zeros((16,), jnp.float32)
                gbuf[0, i, pl.ds(16, 16)] = jnp.zeros((16,), jnp.float32)

            @pl.loop(0, stripe // 128)
            def _(j):
                pltpu.sync_copy(gbuf.at[0],
                                acc.at[pl.ds(s * stripe + j * 128, 128)])

            plsc.subcore_barrier()

            @pl.loop(0, nb)
            def _(blk):
                par = lax.rem(blk, 2)
                pltpu.make_async_copy(
                    src_hbm.at[s].at[pl.ds(blk * bch, bch)],
                    src_i.at[par], semi_s).wait()
                pltpu.make_async_copy(
                    dst_hbm.at[c].at[s].at[pl.ds(blk * bch, bch)],
                    dst_i.at[par], semi_d).wait()

                @pl.when(blk < nb - 1)
                def _():
                    pltpu.async_copy(
                        src_hbm.at[s].at[pl.ds((blk + 1) * bch, bch)],
                        src_i.at[1 - par], semi_s)
                    pltpu.async_copy(
                        dst_hbm.at[c].at[s].at[pl.ds((blk + 1) * bch, bch)],
                        dst_i.at[1 - par], semi_d)

                @pl.loop(0, bch, step=7)
                def _(jj):
                    gs = [pltpu.async_copy(
                              tab_q.at[src_i.at[par].at[jj + b]],
                              gbuf.at[b], semg)
                          for b in range(7)]
                    for b in range(7):
                        gs[b].wait()
                        pltpu.sync_copy(gbuf.at[b],
                                        acc.at[dst_i.at[par].at[jj + b]],
                                        add=True)

            plsc.subcore_barrier()
            pltpu.sync_copy(
                acc.at[pl.ds(s * _WR2, _WR2)],
                out_hbm.at[pl.ds((c * 2 + q) * _NR2 + s * _WR2, _WR2)])

    return msg_kernel(table, src3, dst4)


def kernel(x, edge_index, W1, b1, W2, b2):
    n, d = x.shape
    e = edge_index.shape[1]
    src = edge_index[0]
    dst = edge_index[1]
    nh = n // 2

    ep = -(-e // 8192) * 8192
    pad = ep - e
    srcp = jnp.concatenate([src, jnp.zeros((pad,), jnp.int32)])
    dstp = jnp.concatenate([dst, jnp.full((pad,), n, jnp.int32)])
    src_m = srcp.reshape(_NS, ep // 2048, 128)
    dl0 = jnp.where(dstp < nh, dstp, _NR2)
    dl1 = jnp.where((dstp >= nh) & (dstp < n), dstp - nh, _NR2)
    dst4 = jnp.stack([dl0.reshape(_NS, ep // 2048, 128),
                      dl1.reshape(_NS, ep // 2048, 128)])

    h1 = _tc_matmul(x, W1)
    pdeg = _sc_degree(dst4, n)
    dinv, t1 = _tc_norm(pdeg.reshape(2, _NR2, 16), h1)

    def body(carry, bw):
        table, h = carry
        b_i, w_i = bw
        y = _sc_gather_scatter(table, src_m, dst4, n)
        z, h2, t2 = _tc_step(y.reshape(4, _NR2, 32), h, dinv, b_i, w_i)
        return (t2.reshape(2 * n, 32), h2), z

    bs = jnp.stack([b1.reshape(1, d), b2.reshape(1, d)])
    ws = jnp.stack([W2, W2])
    _, zs = lax.scan(body, (t1.reshape(2 * n, 32), h1), (bs, ws))
    return zs[1]


# speedup vs baseline: 18.3693x; 2.1763x over previous
import functools

import jax
import jax.numpy as jnp
from jax import lax
from jax.experimental import pallas as pl
from jax.experimental.pallas import tpu as pltpu
from jax.experimental.pallas import tpu_sc as plsc

_NC = 2
_NS = 16
_AR2 = 26624
_WR2 = 1568
_NR2 = _NS * _WR2


def _mm_body(x_ref, w_ref, o_ref):
    o_ref[...] = lax.dot_general(
        x_ref[...], w_ref[...], (((1,), (0,)), ((), ())),
        preferred_element_type=jnp.float32, precision=lax.Precision.HIGHEST)


def _tc_matmul(x, w):
    n, d = x.shape
    r = 2000
    return pl.pallas_call(
        _mm_body,
        grid=(n // r,),
        in_specs=[pl.BlockSpec((r, d), lambda i: (i, 0)),
                  pl.BlockSpec((d, d), lambda i: (0, 0))],
        out_specs=pl.BlockSpec((r, d), lambda i: (i, 0)),
        out_shape=jax.ShapeDtypeStruct((n, d), jnp.float32),
    )(x, w)


def _norm_body(p_ref, h_ref, dinv_ref, t_ref):
    deg = p_ref[0, :, 0:1] + 1.0
    dinv = lax.rsqrt(deg)
    dinv_ref[...] = dinv
    h = h_ref[...] * dinv
    t_ref[0, :, :] = h[:, :32]
    t_ref[1, :, :] = h[:, 32:]


def _tc_norm(pdeg, h1):
    n, d = h1.shape
    r = 1000
    hb = (n // 2) // r
    return pl.pallas_call(
        _norm_body,
        grid=(n // r,),
        in_specs=[pl.BlockSpec((1, r, 16), lambda i: (i // hb, i % hb, 0)),
                  pl.BlockSpec((r, d), lambda i: (i, 0))],
        out_specs=(pl.BlockSpec((r, 1), lambda i: (i, 0)),
                   pl.BlockSpec((2, r, 32), lambda i: (0, i, 0))),
        out_shape=(jax.ShapeDtypeStruct((n, 1), jnp.float32),
                   jax.ShapeDtypeStruct((2, n, 32), jnp.float32)),
    )(pdeg, h1)


def _step_body(y_ref, h_ref, dinv_ref, b_ref, w_ref, z_ref, h2_ref, t_ref):
    d = dinv_ref[...]
    y = jnp.concatenate([y_ref[0], y_ref[1]], axis=1)
    z = jnp.maximum(y * d + h_ref[...] * (d * d) + b_ref[...], 0.0)
    z_ref[...] = z
    h2 = lax.dot_general(
        z, w_ref[...], (((1,), (0,)), ((), ())),
        preferred_element_type=jnp.float32, precision=lax.Precision.HIGHEST)
    h2_ref[...] = h2
    ht = h2 * d
    t_ref[0, :, :] = ht[:, :32]
    t_ref[1, :, :] = ht[:, 32:]


def _tc_step(y, h, dinv, b, w):
    n, d = h.shape
    r = 1000
    hb = (n // 2) // r
    return pl.pallas_call(
        _step_body,
        grid=(n // r,),
        in_specs=[pl.BlockSpec((2, r, 32), lambda i: (i // hb, i % hb, 0)),
                  pl.BlockSpec((r, d), lambda i: (i, 0)),
                  pl.BlockSpec((r, 1), lambda i: (i, 0)),
                  pl.BlockSpec((1, d), lambda i: (0, 0)),
                  pl.BlockSpec((d, d), lambda i: (0, 0))],
        out_specs=(pl.BlockSpec((r, d), lambda i: (i, 0)),
                   pl.BlockSpec((r, d), lambda i: (i, 0)),
                   pl.BlockSpec((2, r, 32), lambda i: (0, i, 0))),
        out_shape=(jax.ShapeDtypeStruct((n, d), jnp.float32),
                   jax.ShapeDtypeStruct((n, d), jnp.float32),
                   jax.ShapeDtypeStruct((2, n, 32), jnp.float32)),
    )(y, h, dinv, b, w)


def _sc_degree(dst4, n_nodes):
    k_ch = dst4.shape[2]
    stripe = _AR2 // _NS
    mesh = plsc.VectorSubcoreMesh(core_axis_name="c", subcore_axis_name="s")

    @functools.partial(
        pl.kernel, mesh=mesh,
        compiler_params=pltpu.CompilerParams(use_tc_tiling_on_sc=False),
        out_type=jax.ShapeDtypeStruct((2 * _NR2, 16), jnp.float32),
        scratch_types=[
            pltpu.VMEM((k_ch, 128), jnp.int32),
            pltpu.VMEM((128, 16), jnp.float32),
            pltpu.VMEM((128, 16), jnp.float32),
            pltpu.VMEM_SHARED((_AR2, 16), jnp.float32),
            pltpu.SemaphoreType.DMA,
        ])
    def deg_kernel(dst_hbm, out_hbm, dst_v, ones_v, zero_v, acc, sem):
        c = lax.axis_index("c")
        s = lax.axis_index("s")
        cp = pltpu.async_copy(dst_hbm.at[c].at[s], dst_v, sem)

        @pl.loop(0, 128)
        def _(i):
            ones_v[i, :] = jnp.ones((16,), jnp.float32)
            zero_v[i, :] = jnp.zeros((16,), jnp.float32)

        @pl.loop(0, stripe // 128)
        def _(j):
            pltpu.sync_copy(zero_v, acc.at[pl.ds(s * stripe + j * 128, 128)])

        cp.wait()
        plsc.subcore_barrier()

        @pl.loop(0, k_ch)
        def _(j):
            pltpu.sync_copy(ones_v, acc.at[dst_v.at[j]], add=True)

        plsc.subcore_barrier()
        pltpu.sync_copy(acc.at[pl.ds(s * _WR2, _WR2)],
                        out_hbm.at[pl.ds(c * _NR2 + s * _WR2, _WR2)])

    return deg_kernel(dst4)


def _sc_gather_scatter(table, src3, dst4, n_nodes):
    k_ch = src3.shape[1]
    nb = 8
    bch = k_ch // nb
    stripe = _AR2 // _NS
    mesh = plsc.VectorSubcoreMesh(core_axis_name="c", subcore_axis_name="s")

    @functools.partial(
        pl.kernel, mesh=mesh,
        compiler_params=pltpu.CompilerParams(use_tc_tiling_on_sc=False),
        out_type=jax.ShapeDtypeStruct((4 * _NR2, 32), jnp.float32),
        scratch_types=[
            pltpu.VMEM((2, bch, 128), jnp.int32),
            pltpu.VMEM((2, bch, 128), jnp.int32),
            pltpu.VMEM((7, 128, 32), jnp.float32),
            pltpu.VMEM_SHARED((_AR2, 32), jnp.float32),
            pltpu.SemaphoreType.DMA,
            pltpu.SemaphoreType.DMA,
            pltpu.SemaphoreType.DMA,
        ])
    def msg_kernel(tab_hbm, src_hbm, dst_hbm, out_hbm,
                   src_i, dst_i, gbuf, acc, semi_s, semi_d, semg):
        c = lax.axis_index("c")
        s = lax.axis_index("s")

        for q in range(2):
            if q:
                plsc.subcore_barrier()
            tab_q = tab_hbm.at[pl.ds(q * n_nodes, n_nodes)]

            pltpu.async_copy(src_hbm.at[s].at[pl.ds(0, bch)],
                             src_i.at[0], semi_s)
            pltpu.async_copy(dst_hbm.at[c].at[s].at[pl.ds(0, bch)],
                             dst_i.at[0], semi_d)

            @pl.loop(0, 128)
            def _(i):
                gbuf[0, i, pl.ds(0, 16)] = jnp.zeros((16,), jnp.float32)
                gbuf[0, i, pl.ds(16, 16)] = jnp.zeros((16,), jnp.float32)

            @pl.loop(0, stripe // 128)
            def _(j):
                pltpu.sync_copy(gbuf.at[0],
                                acc.at[pl.ds(s * stripe + j * 128, 128)])

            plsc.subcore_barrier()

            @pl.loop(0, nb)
            def _(blk):
                par = lax.rem(blk, 2)
                pltpu.make_async_copy(
                    src_hbm.at[s].at[pl.ds(blk * bch, bch)],
                    src_i.at[par], semi_s).wait()
                pltpu.make_async_copy(
                    dst_hbm.at[c].at[s].at[pl.ds(blk * bch, bch)],
                    dst_i.at[par], semi_d).wait()

                @pl.when(blk < nb - 1)
                def _():
                    pltpu.async_copy(
                        src_hbm.at[s].at[pl.ds((blk + 1) * bch, bch)],
                        src_i.at[1 - par], semi_s)
                    pltpu.async_copy(
                        dst_hbm.at[c].at[s].at[pl.ds((blk + 1) * bch, bch)],
                        dst_i.at[1 - par], semi_d)

                @pl.loop(0, bch, step=7)
                def _(jj):
                    gs = [pltpu.async_copy(
                              tab_q.at[src_i.at[par].at[jj + b]],
                              gbuf.at[b], semg)
                          for b in range(7)]
                    for b in range(7):
                        gs[b].wait()
                        pltpu.sync_copy(gbuf.at[b],
                                        acc.at[dst_i.at[par].at[jj + b]],
                                        add=True)

            plsc.subcore_barrier()
            pltpu.sync_copy(
                acc.at[pl.ds(s * _WR2, _WR2)],
                out_hbm.at[pl.ds((c * 2 + q) * _NR2 + s * _WR2, _WR2)])

    return msg_kernel(table, src3, dst4)


def kernel(x, edge_index, W1, b1, W2, b2):
    n, d = x.shape
    e = edge_index.shape[1]
    src = edge_index[0]
    dst = edge_index[1]
    nh = n // 2

    ep = -(-e // 8192) * 8192
    pad = ep - e
    srcp = jnp.concatenate([src, jnp.zeros((pad,), jnp.int32)])
    dstp = jnp.concatenate([dst, jnp.full((pad,), n, jnp.int32)])
    src_m = srcp.reshape(_NS, ep // 2048, 128)
    dump = _NR2 + (dstp & 1023)
    dl0 = jnp.where(dstp < nh, dstp, dump)
    dl1 = jnp.where((dstp >= nh) & (dstp < n), dstp - nh, dump)
    dst4 = jnp.stack([dl0.reshape(_NS, ep // 2048, 128),
                      dl1.reshape(_NS, ep // 2048, 128)])

    h1 = _tc_matmul(x, W1)
    pdeg = _sc_degree(dst4, n)
    dinv, t1 = _tc_norm(pdeg.reshape(2, _NR2, 16), h1)

    def body(carry, bw):
        table, h = carry
        b_i, w_i = bw
        y = _sc_gather_scatter(table, src_m, dst4, n)
        z, h2, t2 = _tc_step(y.reshape(4, _NR2, 32), h, dinv, b_i, w_i)
        return (t2.reshape(2 * n, 32), h2), z

    bs = jnp.stack([b1.reshape(1, d), b2.reshape(1, d)])
    ws = jnp.stack([W2, W2])
    _, zs = lax.scan(body, (t1.reshape(2 * n, 32), h1), (bs, ws))
    return zs[1]


# speedup vs baseline: 18.8980x; 1.0288x over previous
import functools

import jax
import jax.numpy as jnp
from jax import lax
from jax.experimental import pallas as pl
from jax.experimental.pallas import tpu as pltpu
from jax.experimental.pallas import tpu_sc as plsc

_NC = 2
_NS = 16
_AR2 = 26624
_WR2 = 1568
_NR2 = _NS * _WR2


def _mm_body(x_ref, w_ref, o_ref):
    o_ref[...] = lax.dot_general(
        x_ref[...], w_ref[...], (((1,), (0,)), ((), ())),
        preferred_element_type=jnp.float32, precision=lax.Precision.HIGHEST)


def _tc_matmul(x, w):
    n, d = x.shape
    r = 2000
    return pl.pallas_call(
        _mm_body,
        grid=(n // r,),
        in_specs=[pl.BlockSpec((r, d), lambda i: (i, 0)),
                  pl.BlockSpec((d, d), lambda i: (0, 0))],
        out_specs=pl.BlockSpec((r, d), lambda i: (i, 0)),
        out_shape=jax.ShapeDtypeStruct((n, d), jnp.float32),
    )(x, w)


def _norm_body(p_ref, h_ref, dinv_ref, t_ref):
    deg = p_ref[0, :, 0:1] + 1.0
    dinv = lax.rsqrt(deg)
    dinv_ref[...] = dinv
    h = h_ref[...] * dinv
    t_ref[0, :, :] = h[:, :32]
    t_ref[1, :, :] = h[:, 32:]


def _tc_norm(pdeg, h1):
    n, d = h1.shape
    r = 1000
    hb = (n // 2) // r
    return pl.pallas_call(
        _norm_body,
        grid=(n // r,),
        in_specs=[pl.BlockSpec((1, r, 16), lambda i: (i // hb, i % hb, 0)),
                  pl.BlockSpec((r, d), lambda i: (i, 0))],
        out_specs=(pl.BlockSpec((r, 1), lambda i: (i, 0)),
                   pl.BlockSpec((2, r, 32), lambda i: (0, i, 0))),
        out_shape=(jax.ShapeDtypeStruct((n, 1), jnp.float32),
                   jax.ShapeDtypeStruct((2, n, 32), jnp.float32)),
    )(pdeg, h1)


def _step_body(y_ref, h_ref, dinv_ref, b_ref, w_ref, z_ref, h2_ref, t_ref):
    d = dinv_ref[...]
    y = jnp.concatenate([y_ref[0], y_ref[1]], axis=1)
    z = jnp.maximum(y * d + h_ref[...] * (d * d) + b_ref[...], 0.0)
    z_ref[...] = z
    h2 = lax.dot_general(
        z, w_ref[...], (((1,), (0,)), ((), ())),
        preferred_element_type=jnp.float32, precision=lax.Precision.HIGHEST)
    h2_ref[...] = h2
    ht = h2 * d
    t_ref[0, :, :] = ht[:, :32]
    t_ref[1, :, :] = ht[:, 32:]


def _tc_step(y, h, dinv, b, w):
    n, d = h.shape
    r = 1000
    hb = (n // 2) // r
    return pl.pallas_call(
        _step_body,
        grid=(n // r,),
        in_specs=[pl.BlockSpec((2, r, 32), lambda i: (i // hb, i % hb, 0)),
                  pl.BlockSpec((r, d), lambda i: (i, 0)),
                  pl.BlockSpec((r, 1), lambda i: (i, 0)),
                  pl.BlockSpec((1, d), lambda i: (0, 0)),
                  pl.BlockSpec((d, d), lambda i: (0, 0))],
        out_specs=(pl.BlockSpec((r, d), lambda i: (i, 0)),
                   pl.BlockSpec((r, d), lambda i: (i, 0)),
                   pl.BlockSpec((2, r, 32), lambda i: (0, i, 0))),
        out_shape=(jax.ShapeDtypeStruct((n, d), jnp.float32),
                   jax.ShapeDtypeStruct((n, d), jnp.float32),
                   jax.ShapeDtypeStruct((2, n, 32), jnp.float32)),
    )(y, h, dinv, b, w)


def _sc_degree(dst4, n_nodes):
    k_ch = dst4.shape[2]
    stripe = _AR2 // _NS
    mesh = plsc.VectorSubcoreMesh(core_axis_name="c", subcore_axis_name="s")

    @functools.partial(
        pl.kernel, mesh=mesh,
        compiler_params=pltpu.CompilerParams(use_tc_tiling_on_sc=False),
        out_type=jax.ShapeDtypeStruct((2 * _NR2, 16), jnp.float32),
        scratch_types=[
            pltpu.VMEM((k_ch, 128), jnp.int32),
            pltpu.VMEM((128, 16), jnp.float32),
            pltpu.VMEM((128, 16), jnp.float32),
            pltpu.VMEM_SHARED((_AR2, 16), jnp.float32),
            pltpu.SemaphoreType.DMA,
        ])
    def deg_kernel(dst_hbm, out_hbm, dst_v, ones_v, zero_v, acc, sem):
        c = lax.axis_index("c")
        s = lax.axis_index("s")
        cp = pltpu.async_copy(dst_hbm.at[c].at[s], dst_v, sem)

        @pl.loop(0, 128)
        def _(i):
            ones_v[i, :] = jnp.ones((16,), jnp.float32)
            zero_v[i, :] = jnp.zeros((16,), jnp.float32)

        @pl.loop(0, stripe // 128)
        def _(j):
            pltpu.sync_copy(zero_v, acc.at[pl.ds(s * stripe + j * 128, 128)])

        cp.wait()
        plsc.subcore_barrier()

        @pl.loop(0, k_ch)
        def _(j):
            pltpu.sync_copy(ones_v, acc.at[dst_v.at[j]], add=True)

        plsc.subcore_barrier()
        pltpu.sync_copy(acc.at[pl.ds(s * _WR2, _WR2)],
                        out_hbm.at[pl.ds(c * _NR2 + s * _WR2, _WR2)])

    return deg_kernel(dst4)


def _sc_gather_scatter(table, src3, dst4, n_nodes):
    k_ch = src3.shape[1]
    nb = 8
    bch = k_ch // nb
    stripe = _AR2 // _NS
    mesh = plsc.VectorSubcoreMesh(core_axis_name="c", subcore_axis_name="s")

    @functools.partial(
        pl.kernel, mesh=mesh,
        compiler_params=pltpu.CompilerParams(use_tc_tiling_on_sc=False),
        out_type=jax.ShapeDtypeStruct((4 * _NR2, 32), jnp.float32),
        scratch_types=[
            pltpu.VMEM((2, bch, 128), jnp.int32),
            pltpu.VMEM((2, bch, 128), jnp.int32),
            pltpu.VMEM((7, 128, 32), jnp.float32),
            pltpu.VMEM_SHARED((_AR2, 32), jnp.float32),
            pltpu.SemaphoreType.DMA,
            pltpu.SemaphoreType.DMA,
            pltpu.SemaphoreType.DMA,
            pltpu.SemaphoreType.DMA,
        ])
    def msg_kernel(tab_hbm, src_hbm, dst_hbm, out_hbm,
                   src_i, dst_i, gbuf, acc, semi_s, semi_d, semg, sems):
        c = lax.axis_index("c")
        s = lax.axis_index("s")

        for q in range(2):
            if q:
                plsc.subcore_barrier()
            tab_q = tab_hbm.at[pl.ds(q * n_nodes, n_nodes)]

            pltpu.async_copy(src_hbm.at[s].at[pl.ds(0, bch)],
                             src_i.at[0], semi_s)
            pltpu.async_copy(dst_hbm.at[c].at[s].at[pl.ds(0, bch)],
                             dst_i.at[0], semi_d)

            @pl.loop(0, 128)
            def _(i):
                gbuf[0, i, pl.ds(0, 16)] = jnp.zeros((16,), jnp.float32)
                gbuf[0, i, pl.ds(16, 16)] = jnp.zeros((16,), jnp.float32)

            @pl.loop(0, stripe // 128)
            def _(j):
                pltpu.sync_copy(gbuf.at[0],
                                acc.at[pl.ds(s * stripe + j * 128, 128)])

            plsc.subcore_barrier()

            @pl.loop(0, nb)
            def _(blk):
                par = lax.rem(blk, 2)
                pltpu.make_async_copy(
                    src_hbm.at[s].at[pl.ds(blk * bch, bch)],
                    src_i.at[par], semi_s).wait()
                pltpu.make_async_copy(
                    dst_hbm.at[c].at[s].at[pl.ds(blk * bch, bch)],
                    dst_i.at[par], semi_d).wait()

                @pl.when(blk < nb - 1)
                def _():
                    pltpu.async_copy(
                        src_hbm.at[s].at[pl.ds((blk + 1) * bch, bch)],
                        src_i.at[1 - par], semi_s)
                    pltpu.async_copy(
                        dst_hbm.at[c].at[s].at[pl.ds((blk + 1) * bch, bch)],
                        dst_i.at[1 - par], semi_d)

                def _drain7():
                    for b in range(7):
                        pltpu.make_async_copy(
                            gbuf.at[b], acc.at[dst_i.at[par].at[0]],
                            sems).wait()

                @pl.loop(0, bch, step=7)
                def _(jj):
                    @pl.when(jj > 0)
                    def _():
                        _drain7()

                    gs = [pltpu.async_copy(
                              tab_q.at[src_i.at[par].at[jj + b]],
                              gbuf.at[b], semg)
                          for b in range(7)]
                    for b in range(7):
                        gs[b].wait()
                        pltpu.async_copy(gbuf.at[b],
                                         acc.at[dst_i.at[par].at[jj + b]],
                                         sems, add=True)

                _drain7()

            plsc.subcore_barrier()
            pltpu.sync_copy(
                acc.at[pl.ds(s * _WR2, _WR2)],
                out_hbm.at[pl.ds((c * 2 + q) * _NR2 + s * _WR2, _WR2)])

    return msg_kernel(table, src3, dst4)


def kernel(x, edge_index, W1, b1, W2, b2):
    n, d = x.shape
    e = edge_index.shape[1]
    src = edge_index[0]
    dst = edge_index[1]
    nh = n // 2

    ep = -(-e // 8192) * 8192
    pad = ep - e
    srcp = jnp.concatenate([src, jnp.zeros((pad,), jnp.int32)])
    dstp = jnp.concatenate([dst, jnp.full((pad,), n, jnp.int32)])
    src_m = srcp.reshape(_NS, ep // 2048, 128)
    dump = _NR2 + (dstp & 1023)
    dl0 = jnp.where(dstp < nh, dstp, dump)
    dl1 = jnp.where((dstp >= nh) & (dstp < n), dstp - nh, dump)
    dst4 = jnp.stack([dl0.reshape(_NS, ep // 2048, 128),
                      dl1.reshape(_NS, ep // 2048, 128)])

    h1 = _tc_matmul(x, W1)
    pdeg = _sc_degree(dst4, n)
    dinv, t1 = _tc_norm(pdeg.reshape(2, _NR2, 16), h1)

    def body(carry, bw):
        table, h = carry
        b_i, w_i = bw
        y = _sc_gather_scatter(table, src_m, dst4, n)
        z, h2, t2 = _tc_step(y.reshape(4, _NR2, 32), h, dinv, b_i, w_i)
        return (t2.reshape(2 * n, 32), h2), z

    bs = jnp.stack([b1.reshape(1, d), b2.reshape(1, d)])
    ws = jnp.stack([W2, W2])
    _, zs = lax.scan(body, (t1.reshape(2 * n, 32), h1), (bs, ws))
    return zs[1]
